# baseline (speedup 1.0000x reference)
import jax
import jax.numpy as jnp
from jax.experimental import pallas as pl
from jax.experimental.pallas import tpu as pltpu

POOL_SIZE = 100000
POOL_DIM = 128
EPS = 1e-8
BLOCK_ROWS = 25000
NUM_BLOCKS = POOL_SIZE // BLOCK_ROWS

_T_DIMS = (((1,), (1,)), ((), ()))
_N_DIMS = (((1,), (0,)), ((), ()))


def _body(x_ref, mem_ref, out_ref, acc_ref):
    i = pl.program_id(0)
    x2 = x_ref[...]
    xnsq = jnp.maximum(jnp.sum(x2 * x2), EPS * EPS)

    m = mem_ref[...]
    dots = jax.lax.dot_general(x2, m, _T_DIMS,
                               preferred_element_type=jnp.float32)
    ones2 = jnp.ones((1, POOL_DIM), jnp.float32)
    m1 = jax.lax.slice(m, (0, 0), (19200, POOL_DIM))
    m2 = jax.lax.slice(m, (19200, 0), (BLOCK_ROWS, POOL_DIM))
    nsq1 = jax.lax.dot_general(ones2, m1 * m1, _T_DIMS,
                               preferred_element_type=jnp.float32)
    nsq2 = jnp.sum(m2 * m2, axis=1).reshape(1, BLOCK_ROWS - 19200)
    nsq = jnp.concatenate([nsq1, nsq2], axis=1)
    sims = dots * jax.lax.rsqrt(jnp.maximum(nsq, EPS * EPS) * xnsq)
    partial = jax.lax.dot_general(sims, m, _N_DIMS,
                                  preferred_element_type=jnp.float32)

    @pl.when(i == 0)
    def _():
        acc_ref[...] = jnp.zeros_like(acc_ref)

    acc_ref[...] += partial

    @pl.when(i == NUM_BLOCKS - 1)
    def _():
        acc = acc_ref[...]
        out_ref[...] = acc / jnp.max(jnp.abs(acc))


def kernel(x, mem):
    out = pl.pallas_call(
        _body,
        grid=(NUM_BLOCKS,),
        in_specs=[
            pl.BlockSpec((1, POOL_DIM), lambda i: (0, 0)),
            pl.BlockSpec((BLOCK_ROWS, POOL_DIM), lambda i: (i, 0)),
        ],
        out_specs=pl.BlockSpec((1, POOL_DIM), lambda i: (0, 0)),
        out_shape=jax.ShapeDtypeStruct((1, POOL_DIM), jnp.float32),
        scratch_shapes=[pltpu.VMEM((1, POOL_DIM), jnp.float32)],
    )(x.reshape(1, POOL_DIM), mem)
    return out.reshape(POOL_DIM)


# speedup vs baseline: 1.0636x; 1.0636x over previous
import jax
import jax.numpy as jnp
from jax.experimental import pallas as pl
from jax.experimental.pallas import tpu as pltpu

POOL_SIZE = 100000
POOL_DIM = 128
EPS = 1e-8
BLOCK_ROWS = 25000
NUM_BLOCKS = POOL_SIZE // BLOCK_ROWS

_T_DIMS = (((1,), (1,)), ((), ()))
_N_DIMS = (((1,), (0,)), ((), ()))


def _body(x_ref, mem_ref, out_ref, acc_ref):
    i = pl.program_id(0)
    x2 = x_ref[...]
    xnsq = jnp.maximum(jnp.sum(x2 * x2), EPS * EPS)

    m = mem_ref[...]
    dots = jax.lax.dot_general(x2, m, _T_DIMS,
                               preferred_element_type=jnp.float32)
    ones2 = jnp.ones((1, POOL_DIM), jnp.float32)
    m1 = jax.lax.slice(m, (0, 0), (8960, POOL_DIM))
    m2 = jax.lax.slice(m, (8960, 0), (BLOCK_ROWS, POOL_DIM))
    nsq1 = jax.lax.dot_general(ones2, m1 * m1, _T_DIMS,
                               preferred_element_type=jnp.float32)
    nsq2 = jnp.sum(m2 * m2, axis=1).reshape(1, BLOCK_ROWS - 8960)
    nsq = jnp.concatenate([nsq1, nsq2], axis=1)
    sims = dots * jax.lax.rsqrt(jnp.maximum(nsq, EPS * EPS) * xnsq)
    partial = jax.lax.dot_general(sims, m, _N_DIMS,
                                  preferred_element_type=jnp.float32)

    @pl.when(i == 0)
    def _():
        acc_ref[...] = jnp.zeros_like(acc_ref)

    acc_ref[...] += partial

    @pl.when(i == NUM_BLOCKS - 1)
    def _():
        acc = acc_ref[...]
        out_ref[...] = acc / jnp.max(jnp.abs(acc))


def kernel(x, mem):
    out = pl.pallas_call(
        _body,
        grid=(NUM_BLOCKS,),
        in_specs=[
            pl.BlockSpec((1, POOL_DIM), lambda i: (0, 0)),
            pl.BlockSpec((BLOCK_ROWS, POOL_DIM), lambda i: (i, 0)),
        ],
        out_specs=pl.BlockSpec((1, POOL_DIM), lambda i: (0, 0)),
        out_shape=jax.ShapeDtypeStruct((1, POOL_DIM), jnp.float32),
        scratch_shapes=[pltpu.VMEM((1, POOL_DIM), jnp.float32)],
    )(x.reshape(1, POOL_DIM), mem)
    return out.reshape(POOL_DIM)


# speedup vs baseline: 1.1242x; 1.0570x over previous
import jax
import jax.numpy as jnp
from jax.experimental import pallas as pl
from jax.experimental.pallas import tpu as pltpu

POOL_SIZE = 100000
POOL_DIM = 128
EPS = 1e-8
BLOCK_ROWS = 25000
NUM_BLOCKS = POOL_SIZE // BLOCK_ROWS

_T_DIMS = (((1,), (1,)), ((), ()))
_N_DIMS = (((1,), (0,)), ((), ()))


def _body(x_ref, mem_ref, out_ref, acc_ref):
    i = pl.program_id(0)
    x2 = x_ref[...]
    xnsq = jnp.maximum(jnp.sum(x2 * x2), EPS * EPS)

    m = mem_ref[...]
    dots = jax.lax.dot_general(x2, m, _T_DIMS,
                               preferred_element_type=jnp.float32)
    ones2 = jnp.ones((1, POOL_DIM), jnp.float32)
    m1 = jax.lax.slice(m, (0, 0), (5120, POOL_DIM))
    m2 = jax.lax.slice(m, (5120, 0), (BLOCK_ROWS, POOL_DIM))
    nsq1 = jax.lax.dot_general(ones2, m1 * m1, _T_DIMS,
                               preferred_element_type=jnp.float32)
    nsq2 = jnp.sum(m2 * m2, axis=1).reshape(1, BLOCK_ROWS - 5120)
    nsq = jnp.concatenate([nsq1, nsq2], axis=1)
    sims = dots * jax.lax.rsqrt(jnp.maximum(nsq, EPS * EPS) * xnsq)
    partial = jax.lax.dot_general(sims, m, _N_DIMS,
                                  preferred_element_type=jnp.float32)

    @pl.when(i == 0)
    def _():
        acc_ref[...] = jnp.zeros_like(acc_ref)

    acc_ref[...] += partial

    @pl.when(i == NUM_BLOCKS - 1)
    def _():
        acc = acc_ref[...]
        out_ref[...] = acc / jnp.max(jnp.abs(acc))


def kernel(x, mem):
    out = pl.pallas_call(
        _body,
        grid=(NUM_BLOCKS,),
        in_specs=[
            pl.BlockSpec((1, POOL_DIM), lambda i: (0, 0)),
            pl.BlockSpec((BLOCK_ROWS, POOL_DIM), lambda i: (i, 0)),
        ],
        out_specs=pl.BlockSpec((1, POOL_DIM), lambda i: (0, 0)),
        out_shape=jax.ShapeDtypeStruct((1, POOL_DIM), jnp.float32),
        scratch_shapes=[pltpu.VMEM((1, POOL_DIM), jnp.float32)],
    )(x.reshape(1, POOL_DIM), mem)
    return out.reshape(POOL_DIM)
